# baseline (speedup 1.0000x reference)
import functools

import jax
import jax.numpy as jnp
from jax import lax
from jax.experimental import pallas as pl
from jax.experimental.pallas import tpu as pltpu
from jax.experimental.pallas import tpu_sc as plsc

VOCAB_N = 100000
HIDDEN_N = 768
MAX_POS_N = 4096
BATCH_N = 4
SEQ_N = 4096

_NC = 2
_NS = 16
_NW = _NC * _NS
_L = 16

_B = BATCH_N * SEQ_N
_PPW = SEQ_N // _NW
_CP = 64
_CR = 32
_NJ = _PPW // _CP
_NK = _NJ * BATCH_N * (_CP // _CR)
_NV = HIDDEN_N // _L


def _scaledsin_table():
    pos = jnp.arange(MAX_POS_N, dtype=jnp.float32)
    half_d = HIDDEN_N // 2
    freq_seq = -jnp.arange(half_d, dtype=jnp.float32) / float(half_d)
    inv_freq = 10000.0 ** freq_seq
    sinusoid = pos[:, None] * inv_freq[None, :]
    return jnp.concatenate([jnp.sin(sinusoid), jnp.cos(sinusoid)], axis=-1)


def _sc_embed(ids_flat, table, posemb, scale16):
    mesh = plsc.VectorSubcoreMesh(core_axis_name="c", subcore_axis_name="s")

    @functools.partial(
        pl.kernel,
        out_type=jax.ShapeDtypeStruct((_B, HIDDEN_N), jnp.float32),
        mesh=mesh,
        scratch_types=[
            pltpu.VMEM((_CR,), jnp.int32),
            pltpu.VMEM((_CR,), jnp.int32),
            pltpu.VMEM((_CR, HIDDEN_N), jnp.float32),
            pltpu.VMEM((_CR, HIDDEN_N), jnp.float32),
            pltpu.VMEM((_CP, HIDDEN_N), jnp.float32),
            pltpu.VMEM((_L,), jnp.float32),
            pltpu.SemaphoreType.DMA,
            pltpu.SemaphoreType.DMA,
            pltpu.SemaphoreType.DMA,
            pltpu.SemaphoreType.DMA,
        ],
    )
    def k(ids_hbm, tab_hbm, pos_hbm, scale_hbm, out_hbm,
          idx0, idx1, rows0, rows1, pos_v, scale_v,
          gsem0, gsem1, osem0, osem1):
        idxb = (idx0, idx1)
        rowsb = (rows0, rows1)
        gsems = (gsem0, gsem1)
        osems = (osem0, osem1)

        wid = lax.axis_index("s") * _NC + lax.axis_index("c")
        pbase = wid * _PPW
        pltpu.sync_copy(scale_hbm, scale_v)
        sv = scale_v[...]

        def chunk_base(kk):
            j = kk // (_NK // _NJ)
            b = lax.rem(kk // (_CP // _CR), BATCH_N)
            h = lax.rem(kk, _CP // _CR)
            return b * SEQ_N + pbase + j * _CP + h * _CR, h

        def issue_gather(kk, i):
            cbase, _ = chunk_base(kk)
            pltpu.sync_copy(ids_hbm.at[pl.ds(cbase, _CR)], idxb[i])
            pltpu.async_copy(tab_hbm.at[idxb[i]], rowsb[i], gsems[i])

        def drain_out(kk, i):
            cbase, _ = chunk_base(kk)
            pltpu.make_async_copy(
                rowsb[i], out_hbm.at[pl.ds(cbase, _CR)], osems[i]).wait()

        issue_gather(0, 0)
        pltpu.sync_copy(pos_hbm.at[pl.ds(pbase, _CP)], pos_v)

        @pl.loop(0, _NK, step=2)
        def _(k0):
            for i in range(2):
                kk = k0 + i
                i2 = 1 - i

                @pl.when(kk + 1 < _NK)
                def _():
                    @pl.when(kk + 1 >= 2)
                    def _():
                        drain_out(kk - 1, i2)
                    issue_gather(kk + 1, i2)

                @pl.when(kk == _NK // _NJ)
                def _():
                    pltpu.sync_copy(pos_hbm.at[pl.ds(pbase + _CP, _CP)], pos_v)

                pltpu.make_async_copy(
                    tab_hbm.at[idxb[i]], rowsb[i], gsems[i]).wait()
                cbase, h = chunk_base(kk)
                hoff = h * _CR

                def row_body(r, c2):
                    for v in range(_NV):
                        sl = pl.ds(v * _L, _L)
                        plsc.addupdate(rowsb[i].at[r, sl],
                                       pos_v[hoff + r, sl] * sv)
                    return c2

                lax.fori_loop(0, _CR, row_body, 0)
                pltpu.async_copy(
                    rowsb[i], out_hbm.at[pl.ds(cbase, _CR)], osems[i])

        drain_out(_NK - 2, 0)
        drain_out(_NK - 1, 1)

    return k(ids_flat, table, posemb, scale16)


def kernel(input_ids, word_embeddings, scale):
    ids_flat = input_ids.reshape(-1).astype(jnp.int32)
    posemb = _scaledsin_table()
    scale16 = jnp.broadcast_to(scale.astype(jnp.float32), (_L,))
    out = _sc_embed(ids_flat, word_embeddings, posemb, scale16)
    return out.reshape(BATCH_N, SEQ_N, HIDDEN_N)


# speedup vs baseline: 1.0399x; 1.0399x over previous
import functools

import jax
import jax.numpy as jnp
from jax import lax
from jax.experimental import pallas as pl
from jax.experimental.pallas import tpu as pltpu
from jax.experimental.pallas import tpu_sc as plsc

VOCAB_N = 100000
HIDDEN_N = 768
MAX_POS_N = 4096
BATCH_N = 4
SEQ_N = 4096

_NC = 2
_NS = 16
_NW = _NC * _NS
_L = 16

_B = BATCH_N * SEQ_N
_PPW = SEQ_N // _NW
_CP = 64
_CR = 32
_NJ = _PPW // _CP
_NH = _CP // _CR
_NK = _NJ * BATCH_N * _NH
_NV = HIDDEN_N // _L


def _scaledsin_table():
    pos = jnp.arange(MAX_POS_N, dtype=jnp.float32)
    half_d = HIDDEN_N // 2
    freq_seq = -jnp.arange(half_d, dtype=jnp.float32) / float(half_d)
    inv_freq = 10000.0 ** freq_seq
    sinusoid = pos[:, None] * inv_freq[None, :]
    return jnp.concatenate([jnp.sin(sinusoid), jnp.cos(sinusoid)], axis=-1)


def _sc_embed(ids_flat, table, posemb, scale16):
    mesh = plsc.VectorSubcoreMesh(core_axis_name="c", subcore_axis_name="s")

    @functools.partial(
        pl.kernel,
        out_type=jax.ShapeDtypeStruct((_B, HIDDEN_N), jnp.float32),
        mesh=mesh,
        scratch_types=[
            pltpu.VMEM((BATCH_N * _PPW,), jnp.int32),
            pltpu.VMEM((_CR, HIDDEN_N), jnp.float32),
            pltpu.VMEM((_CR, HIDDEN_N), jnp.float32),
            pltpu.VMEM((_CP, HIDDEN_N), jnp.float32),
            pltpu.VMEM((_L,), jnp.float32),
            pltpu.SemaphoreType.DMA,
            pltpu.SemaphoreType.DMA,
            pltpu.SemaphoreType.DMA,
            pltpu.SemaphoreType.DMA,
        ],
    )
    def k(ids_hbm, tab_hbm, pos_hbm, scale_hbm, out_hbm,
          idx_all, rows0, rows1, pos_v, scale_v,
          gsem0, gsem1, osem0, osem1):
        rowsb = (rows0, rows1)
        gsems = (gsem0, gsem1)
        osems = (osem0, osem1)

        wid = lax.axis_index("s") * _NC + lax.axis_index("c")
        pbase = wid * _PPW
        pltpu.sync_copy(scale_hbm, scale_v)
        sv = scale_v[...]

        for b in range(BATCH_N):
            pltpu.sync_copy(ids_hbm.at[pl.ds(b * SEQ_N + pbase, _PPW)],
                            idx_all.at[pl.ds(b * _PPW, _PPW)])

        def chunk_dims(kk):
            j = kk // (BATCH_N * _NH)
            b = lax.rem(kk // _NH, BATCH_N)
            h = lax.rem(kk, _NH)
            cbase = b * SEQ_N + pbase + j * _CP + h * _CR
            ioff = b * _PPW + j * _CP + h * _CR
            return cbase, ioff, h

        def issue_gather(kk, i):
            _, ioff, _ = chunk_dims(kk)
            pltpu.async_copy(tab_hbm.at[idx_all.at[pl.ds(ioff, _CR)]],
                             rowsb[i], gsems[i])

        def drain_out(kk, i):
            cbase, _, _ = chunk_dims(kk)
            pltpu.make_async_copy(
                rowsb[i], out_hbm.at[pl.ds(cbase, _CR)], osems[i]).wait()

        issue_gather(0, 0)
        pltpu.sync_copy(pos_hbm.at[pl.ds(pbase, _CP)], pos_v)

        @pl.loop(0, _NK, step=2)
        def _(k0):
            for i in range(2):
                kk = k0 + i
                i2 = 1 - i

                @pl.when(kk + 1 < _NK)
                def _():
                    @pl.when(kk + 1 >= 2)
                    def _():
                        drain_out(kk - 1, i2)
                    issue_gather(kk + 1, i2)

                @pl.when(kk == _NK // _NJ)
                def _():
                    pltpu.sync_copy(pos_hbm.at[pl.ds(pbase + _CP, _CP)],
                                    pos_v)

                _, ioff, _ = chunk_dims(kk)
                pltpu.make_async_copy(
                    tab_hbm.at[idx_all.at[pl.ds(ioff, _CR)]],
                    rowsb[i], gsems[i]).wait()
                cbase, _, h = chunk_dims(kk)
                hoff = h * _CR

                def row_body(r, c2):
                    for v in range(_NV):
                        sl = pl.ds(v * _L, _L)
                        plsc.addupdate(rowsb[i].at[r, sl],
                                       pos_v[hoff + r, sl] * sv)
                    return c2

                lax.fori_loop(0, _CR, row_body, 0)
                pltpu.async_copy(
                    rowsb[i], out_hbm.at[pl.ds(cbase, _CR)], osems[i])

        drain_out(_NK - 2, 0)
        drain_out(_NK - 1, 1)

    return k(ids_flat, table, posemb, scale16)


def kernel(input_ids, word_embeddings, scale):
    ids_flat = input_ids.reshape(-1).astype(jnp.int32)
    posemb = _scaledsin_table()
    scale16 = jnp.broadcast_to(scale.astype(jnp.float32), (_L,))
    out = _sc_embed(ids_flat, word_embeddings, posemb, scale16)
    return out.reshape(BATCH_N, SEQ_N, HIDDEN_N)


# speedup vs baseline: 1.1727x; 1.1277x over previous
import functools

import jax
import jax.numpy as jnp
from jax import lax
from jax.experimental import pallas as pl
from jax.experimental.pallas import tpu as pltpu
from jax.experimental.pallas import tpu_sc as plsc

VOCAB_N = 100000
HIDDEN_N = 768
MAX_POS_N = 4096
BATCH_N = 4
SEQ_N = 4096

_NC = 2
_NS = 16
_NW = _NC * _NS
_L = 16

_B = BATCH_N * SEQ_N
_PPW = SEQ_N // _NW
_CR = 64
_NJ = _PPW // _CR
_NK = _NJ * BATCH_N
_NV = HIDDEN_N // _L


def _scaledsin_table():
    pos = jnp.arange(MAX_POS_N, dtype=jnp.float32)
    half_d = HIDDEN_N // 2
    freq_seq = -jnp.arange(half_d, dtype=jnp.float32) / float(half_d)
    inv_freq = 10000.0 ** freq_seq
    sinusoid = pos[:, None] * inv_freq[None, :]
    return jnp.concatenate([jnp.sin(sinusoid), jnp.cos(sinusoid)], axis=-1)


def _sc_embed(ids_flat, table, posemb, scale16):
    mesh = plsc.VectorSubcoreMesh(core_axis_name="c", subcore_axis_name="s")

    @functools.partial(
        pl.kernel,
        out_type=jax.ShapeDtypeStruct((_B, HIDDEN_N), jnp.float32),
        mesh=mesh,
        scratch_types=[
            pltpu.VMEM((BATCH_N * _PPW,), jnp.int32),
            pltpu.VMEM((_CR, HIDDEN_N), jnp.float32),
            pltpu.VMEM((_CR, HIDDEN_N), jnp.float32),
            pltpu.VMEM((_L,), jnp.float32),
            pltpu.SemaphoreType.DMA,
        ],
    )
    def k(ids_hbm, tab_hbm, pos_hbm, scale_hbm, out_hbm,
          idx_all, rows_v, pos_v, scale_v, sem):
        wid = lax.axis_index("s") * _NC + lax.axis_index("c")
        pbase = wid * _PPW
        pltpu.sync_copy(scale_hbm, scale_v)
        sv = scale_v[...]

        for b in range(BATCH_N):
            pltpu.sync_copy(ids_hbm.at[pl.ds(b * SEQ_N + pbase, _PPW)],
                            idx_all.at[pl.ds(b * _PPW, _PPW)])

        def chunk_body(kk, carry):
            j = kk // BATCH_N
            b = lax.rem(kk, BATCH_N)
            cbase = b * SEQ_N + pbase + j * _CR
            ioff = b * _PPW + j * _CR

            @pl.when(b == 0)
            def _():
                pltpu.sync_copy(pos_hbm.at[pl.ds(pbase + j * _CR, _CR)],
                                pos_v)

            pltpu.async_copy(tab_hbm.at[idx_all.at[pl.ds(ioff, _CR)]],
                             rows_v, sem).wait()

            def row_body(r, c2):
                for v in range(_NV):
                    sl = pl.ds(v * _L, _L)
                    plsc.addupdate(rows_v.at[r, sl], pos_v[r, sl] * sv)
                return c2

            lax.fori_loop(0, _CR, row_body, 0)
            pltpu.sync_copy(rows_v, out_hbm.at[pl.ds(cbase, _CR)])
            return carry

        lax.fori_loop(0, _NK, chunk_body, 0)

    return k(ids_flat, table, posemb, scale16)


def kernel(input_ids, word_embeddings, scale):
    ids_flat = input_ids.reshape(-1).astype(jnp.int32)
    posemb = _scaledsin_table()
    scale16 = jnp.broadcast_to(scale.astype(jnp.float32), (_L,))
    out = _sc_embed(ids_flat, word_embeddings, posemb, scale16)
    return out.reshape(BATCH_N, SEQ_N, HIDDEN_N)


# speedup vs baseline: 1.3679x; 1.1664x over previous
import functools

import jax
import jax.numpy as jnp
import numpy as np
from jax import lax
from jax.experimental import pallas as pl
from jax.experimental.pallas import tpu as pltpu
from jax.experimental.pallas import tpu_sc as plsc

VOCAB_N = 100000
HIDDEN_N = 768
MAX_POS_N = 4096
BATCH_N = 4
SEQ_N = 4096

_NC = 2
_NS = 16
_NW = _NC * _NS
_L = 16

_B = BATCH_N * SEQ_N
_PPW = SEQ_N // _NW
_CR = 64
_NJ = _PPW // _CR
_NK = _NJ * BATCH_N
_NV = HIDDEN_N // _L


def _scaledsin_table():
    pos = np.arange(MAX_POS_N, dtype=np.float32)
    half_d = HIDDEN_N // 2
    freq_seq = -np.arange(half_d, dtype=np.float32) / np.float32(half_d)
    inv_freq = (np.float32(10000.0) ** freq_seq).astype(np.float32)
    sinusoid = pos[:, None] * inv_freq[None, :]
    tab = np.concatenate([np.sin(sinusoid), np.cos(sinusoid)], axis=-1)
    return jnp.asarray(tab.astype(np.float32))


def _sc_embed(ids_flat, table, posemb, scale16):
    mesh = plsc.VectorSubcoreMesh(core_axis_name="c", subcore_axis_name="s")

    @functools.partial(
        pl.kernel,
        out_type=jax.ShapeDtypeStruct((_B, HIDDEN_N), jnp.float32),
        mesh=mesh,
        scratch_types=[
            pltpu.VMEM((BATCH_N * _PPW,), jnp.int32),
            pltpu.VMEM((_CR, HIDDEN_N), jnp.float32),
            pltpu.VMEM((_CR, HIDDEN_N), jnp.float32),
            pltpu.VMEM((_L,), jnp.float32),
            pltpu.SemaphoreType.DMA,
        ],
    )
    def k(ids_hbm, tab_hbm, pos_hbm, scale_hbm, out_hbm,
          idx_all, rows_v, pos_v, scale_v, sem):
        wid = lax.axis_index("s") * _NC + lax.axis_index("c")
        pbase = wid * _PPW
        pltpu.sync_copy(scale_hbm, scale_v)
        sv = scale_v[...]

        for b in range(BATCH_N):
            pltpu.sync_copy(ids_hbm.at[pl.ds(b * SEQ_N + pbase, _PPW)],
                            idx_all.at[pl.ds(b * _PPW, _PPW)])

        def chunk_body(kk, carry):
            j = kk // BATCH_N
            b = lax.rem(kk, BATCH_N)
            cbase = b * SEQ_N + pbase + j * _CR
            ioff = b * _PPW + j * _CR

            @pl.when(b == 0)
            def _():
                pltpu.sync_copy(pos_hbm.at[pl.ds(pbase + j * _CR, _CR)],
                                pos_v)

            pltpu.async_copy(tab_hbm.at[idx_all.at[pl.ds(ioff, _CR)]],
                             rows_v, sem).wait()

            def row_body(r, c2):
                for v in range(_NV):
                    sl = pl.ds(v * _L, _L)
                    plsc.addupdate(rows_v.at[r, sl], pos_v[r, sl] * sv)
                return c2

            lax.fori_loop(0, _CR, row_body, 0)
            pltpu.sync_copy(rows_v, out_hbm.at[pl.ds(cbase, _CR)])
            return carry

        lax.fori_loop(0, _NK, chunk_body, 0)

    return k(ids_flat, table, posemb, scale16)


def kernel(input_ids, word_embeddings, scale):
    ids_flat = input_ids.reshape(-1).astype(jnp.int32)
    posemb = _scaledsin_table()
    scale16 = jnp.broadcast_to(scale.astype(jnp.float32), (_L,))
    out = _sc_embed(ids_flat, word_embeddings, posemb, scale16)
    return out.reshape(BATCH_N, SEQ_N, HIDDEN_N)
